# baseline (speedup 1.0000x reference)
import jax
import jax.numpy as jnp
from jax import lax
from jax.experimental import pallas as pl
from jax.experimental.pallas import tpu as pltpu
from jax.experimental.pallas import tpu_sc as plsc

N_NODES = 10000
N_EDGES = 160000
N_SPEC = 4
N_TILES = 32
EDGES_PER_TILE = 5120
CHUNK = 128
N_CHUNKS = EDGES_PER_TILE // CHUNK
GROUPS = CHUNK // 16
E_PAD = N_TILES * EDGES_PER_TILE

NROWS = 10240
ACC_ROWS = NROWS * N_SPEC
ROWS_PER_TILE = ACC_ROWS // 16
NT_COLS = 8
TCOLS = 24

CUTOFF = 5.0
NU = 0.1
MP = 0.1

SH0 = 0.28209479177387814 * NU
SH1 = 0.4886025119029199 * NU
SH2A = 1.0925484305920792 * NU
SH2C = 0.31539156525252005 * NU
SH2E = 0.5462742152960396 * NU

RC_53 = 5.0 / 3.0
RC_103 = 10.0 / 3.0
RC_25 = 2.5
RC_5 = 5.0


def _sc_body(nt_hbm, cen_hbm, nbr_hbm, out_hbm,
             cen_v, nbr_v, ndc0, ndn0, ndc1, ndn1, msg0, msg1, idx0, idx1,
             acc_s, sg0, sg1, ss0, ss1):
    cid = lax.axis_index("c")
    sid = lax.axis_index("s")
    wid = sid * 2 + cid
    bufs = ((ndc0, ndn0, msg0, idx0, sg0, ss0),
            (ndc1, ndn1, msg1, idx1, sg1, ss1))

    pltpu.sync_copy(cen_hbm.at[wid], cen_v)
    pltpu.sync_copy(nbr_hbm.at[wid], nbr_v)

    zero16 = jnp.zeros((16,), jnp.float32)

    def _zrow(rr, carry):
        msg0[rr, pl.ds(0, 16)] = zero16
        msg0[rr, pl.ds(TCOLS - 16, 16)] = zero16
        msg1[rr, pl.ds(0, 16)] = zero16
        msg1[rr, pl.ds(TCOLS - 16, 16)] = zero16
        return carry

    lax.fori_loop(0, CHUNK, _zrow, 0)
    for z in range(ROWS_PER_TILE // CHUNK):
        pltpu.sync_copy(
            msg0, acc_s.at[pl.ds(sid * ROWS_PER_TILE + z * CHUNK, CHUNK)])
    plsc.subcore_barrier()

    lane = lax.iota(jnp.int32, 16)

    def _start_gathers(g, p):
        ndc_v, ndn_v, _, _, sg, _ = bufs[p]
        pltpu.async_copy(nt_hbm.at[cen_v.at[g]], ndc_v, sg)
        pltpu.async_copy(nt_hbm.at[nbr_v.at[g]], ndn_v, sg)

    def _wait_gathers(p):
        ndc_v, ndn_v, _, _, sg, _ = bufs[p]
        pltpu.make_async_copy(nt_hbm.at[cen_v.at[0]], ndc_v, sg).wait()
        pltpu.make_async_copy(nt_hbm.at[nbr_v.at[0]], ndn_v, sg).wait()

    def _start_scatter(p):
        _, _, msg_v, idx_v, _, ss = bufs[p]
        pltpu.async_copy(msg_v, acc_s.at[idx_v], ss, add=True)

    def _wait_scatter(p):
        _, _, msg_v, idx_v, _, ss = bufs[p]
        pltpu.make_async_copy(msg_v, acc_s.at[idx_v], ss).wait()

    def _compute_chunk(g, ndc_v, ndn_v, msg_v, idx_v):
        def grp(j, carry):
            row16 = lane + j * 16
            gv = jnp.full((16,), g, jnp.int32)
            c0 = jnp.zeros((16,), jnp.int32)
            cen16 = plsc.load_gather(cen_v, [gv, row16])
            pcx = plsc.load_gather(ndc_v, [row16, c0])
            pcy = plsc.load_gather(ndc_v, [row16, c0 + 1])
            pcz = plsc.load_gather(ndc_v, [row16, c0 + 2])
            pnx = plsc.load_gather(ndn_v, [row16, c0])
            pny = plsc.load_gather(ndn_v, [row16, c0 + 1])
            pnz = plsc.load_gather(ndn_v, [row16, c0 + 2])
            spf = plsc.load_gather(ndn_v, [row16, c0 + 3])

            vx = pnx - pcx
            vy = pny - pcy
            vz = pnz - pcz
            s = vx * vx + vy * vy + vz * vz + 1e-12

            bits = plsc.bitcast(s, jnp.int32)
            y = plsc.bitcast(jnp.int32(0x5F3759DF) - (bits >> 1),
                             jnp.float32)
            hs = 0.5 * s
            for _ in range(3):
                y = y * (1.5 - hs * y * y)
            r = s * y
            ux = vx * y
            uy = vy * y
            uz = vz * y

            xc = jnp.minimum(r * (1.0 / CUTOFF), 1.0)
            u = xc - 0.5
            t = u * u
            sinpu = u * (3.141592653589793 + t * (-5.16771278004997
                    + t * (2.5501640398773455 + t * (-0.5992645293207921
                    + t * (0.08214588661112823
                    + t * -0.007370430945714351)))))
            fc = 0.5 - 0.5 * sinpu

            g0 = jnp.exp(-(r * r)) * fc
            g53 = jnp.exp(-((r - RC_53) * (r - RC_53))) * fc
            g103 = jnp.exp(-((r - RC_103) * (r - RC_103))) * fc
            g25 = jnp.exp(-((r - RC_25) * (r - RC_25))) * fc
            g5 = jnp.exp(-((r - RC_5) * (r - RC_5))) * fc

            sh1 = (uy * SH1, uz * SH1, ux * SH1)
            sh2 = (ux * uy * SH2A, uy * uz * SH2A,
                   (3.0 * uz * uz - 1.0) * SH2C,
                   ux * uz * SH2A, (ux * ux - uy * uy) * SH2E)

            terms = [g0 * SH0, g53 * SH0, g103 * SH0, g5 * SH0]
            rb1 = (g0, g25, g5)
            for m in range(3):
                for n in range(3):
                    terms.append(sh1[m] * rb1[n])
            rb2 = (g0, g5)
            for m in range(5):
                for n in range(2):
                    terms.append(sh2[m] * rb2[n])

            for k, val in enumerate(terms):
                col = jnp.full((16,), k, jnp.int32)
                plsc.store_scatter(msg_v, [row16, col], val)

            idx_v[pl.ds(j * 16, 16)] = spf.astype(jnp.int32) * NROWS + cen16
            return carry

        lax.fori_loop(0, GROUPS, grp, 0)

    _start_gathers(0, 0)

    def _phase(g, p):
        @pl.when(g + 1 < N_CHUNKS)
        def _():
            _start_gathers(g + 1, 1 - p)

        @pl.when(g >= 2)
        def _():
            _wait_scatter(p)

        _wait_gathers(p)
        _compute_chunk(g, bufs[p][0], bufs[p][1], bufs[p][2], bufs[p][3])
        _start_scatter(p)

    def _pipe(g2, carry):
        _phase(g2 * 2, 0)
        _phase(g2 * 2 + 1, 1)
        return carry

    lax.fori_loop(0, N_CHUNKS // 2, _pipe, 0)
    _wait_scatter(0)
    _wait_scatter(1)

    plsc.subcore_barrier()
    rows = pl.ds(sid * ROWS_PER_TILE, ROWS_PER_TILE)
    pltpu.sync_copy(acc_s.at[rows], out_hbm.at[cid, rows, :])


def _sc_call(node_tab, centers, neighbors):
    mesh = plsc.VectorSubcoreMesh(core_axis_name="c", subcore_axis_name="s")
    return pl.kernel(
        _sc_body,
        out_type=jax.ShapeDtypeStruct((2, ACC_ROWS, TCOLS), jnp.float32),
        mesh=mesh,
        compiler_params=pltpu.CompilerParams(
            needs_layout_passes=False, use_tc_tiling_on_sc=False),
        scratch_types=[
            pltpu.VMEM((N_CHUNKS, CHUNK), jnp.int32),
            pltpu.VMEM((N_CHUNKS, CHUNK), jnp.int32),
            pltpu.VMEM((CHUNK, NT_COLS), jnp.float32),
            pltpu.VMEM((CHUNK, NT_COLS), jnp.float32),
            pltpu.VMEM((CHUNK, NT_COLS), jnp.float32),
            pltpu.VMEM((CHUNK, NT_COLS), jnp.float32),
            pltpu.VMEM((CHUNK, TCOLS), jnp.float32),
            pltpu.VMEM((CHUNK, TCOLS), jnp.float32),
            pltpu.VMEM((CHUNK,), jnp.int32),
            pltpu.VMEM((CHUNK,), jnp.int32),
            pltpu.VMEM_SHARED((ACC_ROWS, TCOLS), jnp.float32),
            pltpu.SemaphoreType.DMA,
            pltpu.SemaphoreType.DMA,
            pltpu.SemaphoreType.DMA,
            pltpu.SemaphoreType.DMA,
        ],
    )(node_tab, centers, neighbors)


def _tc_body(p_ref, w_ref, w1_ref, b1_ref, w2_ref, b2_ref, o_ref):
    a = p_ref[0] + p_ref[1]
    feat = sum(
        jnp.dot(a[s], w_ref[s], preferred_element_type=jnp.float32,
                precision=jax.lax.Precision.HIGHEST)
        for s in range(N_SPEC)) * MP
    sq = feat * feat
    inv0 = sq[:, 0:32]
    inv1 = sq[:, 32:56] + sq[:, 56:80] + sq[:, 80:104]
    inv2 = (sq[:, 104:120] + sq[:, 120:136] + sq[:, 136:152]
            + sq[:, 152:168] + sq[:, 168:184])
    inv = jnp.concatenate([inv0, inv1, inv2], axis=1)
    h = jnp.maximum(
        jnp.dot(inv, w1_ref[...], preferred_element_type=jnp.float32)
        + b1_ref[...], 0.0)
    o_ref[...] = (jnp.dot(h, w2_ref[...], preferred_element_type=jnp.float32)
                  + b2_ref[...])


def _tc_head(p, W, W1, b1, W2, b2):
    blk = 1024
    return pl.pallas_call(
        _tc_body,
        grid=(NROWS // blk,),
        in_specs=[
            pl.BlockSpec((2, N_SPEC, blk, TCOLS), lambda i: (0, 0, i, 0)),
            pl.BlockSpec((N_SPEC, TCOLS, 184), lambda i: (0, 0, 0)),
            pl.BlockSpec((72, 64), lambda i: (0, 0)),
            pl.BlockSpec((1, 64), lambda i: (0, 0)),
            pl.BlockSpec((64, 1), lambda i: (0, 0)),
            pl.BlockSpec((1, 1), lambda i: (0, 0)),
        ],
        out_specs=pl.BlockSpec((blk, 1), lambda i: (i, 0)),
        out_shape=jax.ShapeDtypeStruct((NROWS, 1), jnp.float32),
    )(p, W, W1, b1, W2, b2)


def _term_onehot():
    import numpy as np
    oh = np.zeros((TCOLS, 8, 184), np.float32)
    k = 0
    for l, (nl, base) in enumerate(((4, 0), (3, 32), (2, 104))):
        for m in range(2 * l + 1):
            for n in range(nl):
                for c in range(8):
                    oh[k, c, base + m * 8 * nl + c * nl + n] = 1.0
                k += 1
    return jnp.asarray(oh)


def kernel(positions, embeddings, W1, b1, W2, b2, species, pairs):
    pp = jnp.pad(pairs.astype(jnp.int32), ((0, E_PAD - N_EDGES), (0, 0)),
                 constant_values=N_NODES)
    centers = pp[:, 0].reshape(N_TILES, N_CHUNKS, CHUNK)
    neighbors = pp[:, 1].reshape(N_TILES, N_CHUNKS, CHUNK)

    node_tab = jnp.pad(
        jnp.concatenate(
            [positions, species.astype(jnp.float32)[:, None]], axis=1),
        ((0, NROWS - N_NODES), (0, NT_COLS - 4)))

    p = _sc_call(node_tab, centers, neighbors)
    p = p.reshape(2, N_SPEC, NROWS, TCOLS)
    W = jnp.einsum('sc,kcj->skj', embeddings, _term_onehot(),
                   precision=jax.lax.Precision.HIGHEST)
    out = _tc_head(p, W, W1, b1.reshape(1, 64), W2, b2.reshape(1, 1))
    return out[:N_NODES]


# speedup vs baseline: 1.3136x; 1.3136x over previous
import jax
import jax.numpy as jnp
from jax import lax
from jax.experimental import pallas as pl
from jax.experimental.pallas import tpu as pltpu
from jax.experimental.pallas import tpu_sc as plsc

N_NODES = 10000
N_EDGES = 160000
N_SPEC = 4
N_TILES = 32
EDGES_PER_TILE = 5120
CHUNK = 128
N_CHUNKS = EDGES_PER_TILE // CHUNK
GROUPS = CHUNK // 16
E_PAD = N_TILES * EDGES_PER_TILE

NROWS = 10240
ACC_ROWS = NROWS * N_SPEC
ROWS_PER_TILE = ACC_ROWS // 16
NT_COLS = 8
TCOLS = 24

CUTOFF = 5.0
NU = 0.1
MP = 0.1

SH0 = 0.28209479177387814 * NU
SH1 = 0.4886025119029199 * NU
SH2A = 1.0925484305920792 * NU
SH2C = 0.31539156525252005 * NU
SH2E = 0.5462742152960396 * NU

RC_53 = 5.0 / 3.0
RC_103 = 10.0 / 3.0
RC_25 = 2.5
RC_5 = 5.0


def _sc_body(nt_hbm, cen_hbm, nbr_hbm, out_hbm,
             cen_v, nbr_v, ndc0, ndn0, ndc1, ndn1, msg0, msg1, idx0, idx1,
             acc_s, sg0, sg1, ss0, ss1):
    cid = lax.axis_index("c")
    sid = lax.axis_index("s")
    wid = sid * 2 + cid
    bufs = ((ndc0, ndn0, msg0, idx0, sg0, ss0),
            (ndc1, ndn1, msg1, idx1, sg1, ss1))

    pltpu.sync_copy(cen_hbm.at[wid], cen_v)
    pltpu.sync_copy(nbr_hbm.at[wid], nbr_v)

    zero16 = jnp.zeros((16,), jnp.float32)

    def _zrow(rr, carry):
        msg0[rr, pl.ds(0, 16)] = zero16
        msg0[rr, pl.ds(TCOLS - 16, 16)] = zero16
        msg1[rr, pl.ds(0, 16)] = zero16
        msg1[rr, pl.ds(TCOLS - 16, 16)] = zero16
        return carry

    lax.fori_loop(0, CHUNK, _zrow, 0)
    for z in range(ROWS_PER_TILE // CHUNK):
        pltpu.sync_copy(
            msg0, acc_s.at[pl.ds(sid * ROWS_PER_TILE + z * CHUNK, CHUNK)])
    plsc.subcore_barrier()

    lane = lax.iota(jnp.int32, 16)

    def _start_gathers(g, p):
        ndc_v, ndn_v, _, _, sg, _ = bufs[p]
        pltpu.async_copy(nt_hbm.at[cen_v.at[g]], ndc_v, sg)
        pltpu.async_copy(nt_hbm.at[nbr_v.at[g]], ndn_v, sg)

    def _wait_gathers(p):
        ndc_v, ndn_v, _, _, sg, _ = bufs[p]
        pltpu.make_async_copy(nt_hbm.at[cen_v.at[0]], ndc_v, sg).wait()
        pltpu.make_async_copy(nt_hbm.at[nbr_v.at[0]], ndn_v, sg).wait()

    def _start_scatter(p):
        _, _, msg_v, idx_v, _, ss = bufs[p]
        pltpu.async_copy(msg_v, acc_s.at[idx_v], ss, add=True)

    def _wait_scatter(p):
        _, _, msg_v, idx_v, _, ss = bufs[p]
        pltpu.make_async_copy(msg_v, acc_s.at[idx_v], ss).wait()

    def _compute_chunk(g, ndc_v, ndn_v, msg_v, idx_v):
        def grp(j, carry):
            row16 = lane + j * 16
            gv = jnp.full((16,), g, jnp.int32)
            c0 = jnp.zeros((16,), jnp.int32)
            cen16 = plsc.load_gather(cen_v, [gv, row16])
            pcx = plsc.load_gather(ndc_v, [row16, c0])
            pcy = plsc.load_gather(ndc_v, [row16, c0 + 1])
            pcz = plsc.load_gather(ndc_v, [row16, c0 + 2])
            pnx = plsc.load_gather(ndn_v, [row16, c0])
            pny = plsc.load_gather(ndn_v, [row16, c0 + 1])
            pnz = plsc.load_gather(ndn_v, [row16, c0 + 2])
            spf = plsc.load_gather(ndn_v, [row16, c0 + 3])

            vx = pnx - pcx
            vy = pny - pcy
            vz = pnz - pcz
            s = vx * vx + vy * vy + vz * vz + 1e-12

            bits = plsc.bitcast(s, jnp.int32)
            y = plsc.bitcast(jnp.int32(0x5F3759DF) - (bits >> 1),
                             jnp.float32)
            hs = 0.5 * s
            for _ in range(3):
                y = y * (1.5 - hs * y * y)
            r = s * y
            ux = vx * y
            uy = vy * y
            uz = vz * y

            xc = jnp.minimum(r * (1.0 / CUTOFF), 1.0)
            u = xc - 0.5
            t = u * u
            sinpu = u * (3.141592653589793 + t * (-5.16771278004997
                    + t * (2.5501640398773455 + t * (-0.5992645293207921
                    + t * (0.08214588661112823
                    + t * -0.007370430945714351)))))
            fc = 0.5 - 0.5 * sinpu

            g0 = jnp.exp(-(r * r)) * fc
            g53 = jnp.exp(-((r - RC_53) * (r - RC_53))) * fc
            g103 = jnp.exp(-((r - RC_103) * (r - RC_103))) * fc
            g25 = jnp.exp(-((r - RC_25) * (r - RC_25))) * fc
            g5 = jnp.exp(-((r - RC_5) * (r - RC_5))) * fc

            sh1 = (uy * SH1, uz * SH1, ux * SH1)
            sh2 = (ux * uy * SH2A, uy * uz * SH2A,
                   (3.0 * uz * uz - 1.0) * SH2C,
                   ux * uz * SH2A, (ux * ux - uy * uy) * SH2E)

            terms = [g0 * SH0, g53 * SH0, g103 * SH0, g5 * SH0]
            rb1 = (g0, g25, g5)
            for m in range(3):
                for n in range(3):
                    terms.append(sh1[m] * rb1[n])
            rb2 = (g0, g5)
            for m in range(5):
                for n in range(2):
                    terms.append(sh2[m] * rb2[n])

            for k, val in enumerate(terms):
                col = jnp.full((16,), k, jnp.int32)
                plsc.store_scatter(msg_v, [row16, col], val)

            idx_v[pl.ds(j * 16, 16)] = cen16 * N_SPEC + spf.astype(jnp.int32)
            return carry

        lax.fori_loop(0, GROUPS, grp, 0)

    _start_gathers(0, 0)

    def _phase(g, p):
        @pl.when(g + 1 < N_CHUNKS)
        def _():
            _start_gathers(g + 1, 1 - p)

        @pl.when(g >= 2)
        def _():
            _wait_scatter(p)

        _wait_gathers(p)
        _compute_chunk(g, bufs[p][0], bufs[p][1], bufs[p][2], bufs[p][3])
        _start_scatter(p)

    def _pipe(g2, carry):
        _phase(g2 * 2, 0)
        _phase(g2 * 2 + 1, 1)
        return carry

    lax.fori_loop(0, N_CHUNKS // 2, _pipe, 0)
    _wait_scatter(0)
    _wait_scatter(1)

    plsc.subcore_barrier()
    rows = pl.ds(sid * ROWS_PER_TILE, ROWS_PER_TILE)
    pltpu.sync_copy(acc_s.at[rows], out_hbm.at[cid, rows, :])


def _sc_call(node_tab, centers, neighbors):
    mesh = plsc.VectorSubcoreMesh(core_axis_name="c", subcore_axis_name="s")
    return pl.kernel(
        _sc_body,
        out_type=jax.ShapeDtypeStruct((2, ACC_ROWS, TCOLS), jnp.float32),
        mesh=mesh,
        compiler_params=pltpu.CompilerParams(
            needs_layout_passes=False, use_tc_tiling_on_sc=False),
        scratch_types=[
            pltpu.VMEM((N_CHUNKS, CHUNK), jnp.int32),
            pltpu.VMEM((N_CHUNKS, CHUNK), jnp.int32),
            pltpu.VMEM((CHUNK, NT_COLS), jnp.float32),
            pltpu.VMEM((CHUNK, NT_COLS), jnp.float32),
            pltpu.VMEM((CHUNK, NT_COLS), jnp.float32),
            pltpu.VMEM((CHUNK, NT_COLS), jnp.float32),
            pltpu.VMEM((CHUNK, TCOLS), jnp.float32),
            pltpu.VMEM((CHUNK, TCOLS), jnp.float32),
            pltpu.VMEM((CHUNK,), jnp.int32),
            pltpu.VMEM((CHUNK,), jnp.int32),
            pltpu.VMEM_SHARED((ACC_ROWS, TCOLS), jnp.float32),
            pltpu.SemaphoreType.DMA,
            pltpu.SemaphoreType.DMA,
            pltpu.SemaphoreType.DMA,
            pltpu.SemaphoreType.DMA,
        ],
    )(node_tab, centers, neighbors)


def _tc_body(p_ref, w_ref, w1_ref, b1_ref, w2_ref, b2_ref, o_ref):
    a = p_ref[0] + p_ref[1]
    feat = jnp.dot(a, w_ref[...], preferred_element_type=jnp.float32,
                   precision=jax.lax.Precision.HIGHEST) * MP
    sq = feat * feat
    inv0 = sq[:, 0:32]
    inv1 = sq[:, 32:56] + sq[:, 56:80] + sq[:, 80:104]
    inv2 = (sq[:, 104:120] + sq[:, 120:136] + sq[:, 136:152]
            + sq[:, 152:168] + sq[:, 168:184])
    inv = jnp.concatenate([inv0, inv1, inv2], axis=1)
    h = jnp.maximum(
        jnp.dot(inv, w1_ref[...], preferred_element_type=jnp.float32)
        + b1_ref[...], 0.0)
    o_ref[...] = (jnp.dot(h, w2_ref[...], preferred_element_type=jnp.float32)
                  + b2_ref[...])


def _tc_head(p, W, W1, b1, W2, b2):
    blk = 400
    return pl.pallas_call(
        _tc_body,
        grid=(N_NODES // blk,),
        in_specs=[
            pl.BlockSpec((2, blk, N_SPEC * TCOLS), lambda i: (0, i, 0)),
            pl.BlockSpec((N_SPEC * TCOLS, 184), lambda i: (0, 0)),
            pl.BlockSpec((72, 64), lambda i: (0, 0)),
            pl.BlockSpec((1, 64), lambda i: (0, 0)),
            pl.BlockSpec((64, 1), lambda i: (0, 0)),
            pl.BlockSpec((1, 1), lambda i: (0, 0)),
        ],
        out_specs=pl.BlockSpec((blk, 1), lambda i: (i, 0)),
        out_shape=jax.ShapeDtypeStruct((N_NODES, 1), jnp.float32),
    )(p, W, W1, b1, W2, b2)


def _term_onehot():
    import numpy as np
    oh = np.zeros((TCOLS, 8, 184), np.float32)
    k = 0
    for l, (nl, base) in enumerate(((4, 0), (3, 32), (2, 104))):
        for m in range(2 * l + 1):
            for n in range(nl):
                for c in range(8):
                    oh[k, c, base + m * 8 * nl + c * nl + n] = 1.0
                k += 1
    return jnp.asarray(oh)


def kernel(positions, embeddings, W1, b1, W2, b2, species, pairs):
    pp = jnp.pad(pairs.astype(jnp.int32), ((0, E_PAD - N_EDGES), (0, 0)),
                 constant_values=N_NODES)
    centers = pp[:, 0].reshape(N_TILES, N_CHUNKS, CHUNK)
    neighbors = pp[:, 1].reshape(N_TILES, N_CHUNKS, CHUNK)

    node_tab = jnp.pad(
        jnp.concatenate(
            [positions, species.astype(jnp.float32)[:, None]], axis=1),
        ((0, NROWS - N_NODES), (0, NT_COLS - 4)))

    p = _sc_call(node_tab, centers, neighbors)
    p = p.reshape(2, NROWS, N_SPEC * TCOLS)
    W = jnp.einsum('sc,kcj->skj', embeddings, _term_onehot(),
                   precision=jax.lax.Precision.HIGHEST)
    W = W.reshape(N_SPEC * TCOLS, 184)
    return _tc_head(p, W, W1, b1.reshape(1, 64), W2, b2.reshape(1, 1))


# speedup vs baseline: 1.3561x; 1.0324x over previous
import jax
import jax.numpy as jnp
from jax import lax
from jax.experimental import pallas as pl
from jax.experimental.pallas import tpu as pltpu
from jax.experimental.pallas import tpu_sc as plsc

N_NODES = 10000
N_EDGES = 160000
N_SPEC = 4
N_TILES = 32
EDGES_PER_TILE = 5120
CHUNK = 128
N_CHUNKS = EDGES_PER_TILE // CHUNK
GROUPS = CHUNK // 16
E_PAD = N_TILES * EDGES_PER_TILE

NROWS = 10240
ACC_ROWS = NROWS * N_SPEC
ROWS_PER_TILE = ACC_ROWS // 16
NT_COLS = 8
TCOLS = 24

CUTOFF = 5.0
NU = 0.1
MP = 0.1

SH0 = 0.28209479177387814 * NU
SH1 = 0.4886025119029199 * NU
SH2A = 1.0925484305920792 * NU
SH2C = 0.31539156525252005 * NU
SH2E = 0.5462742152960396 * NU

RC_53 = 5.0 / 3.0
RC_103 = 10.0 / 3.0
RC_25 = 2.5
RC_5 = 5.0


def _sc_body(nt_hbm, cen_hbm, nbr_hbm, out_hbm,
             cen_v, nbr_v, ndc0, ndn0, ndc1, ndn1, msg0, msg1, idx0, idx1,
             acc_s, sg0, sg1, ss0, ss1):
    cid = lax.axis_index("c")
    sid = lax.axis_index("s")
    wid = sid * 2 + cid
    bufs = ((ndc0, ndn0, msg0, idx0, sg0, ss0),
            (ndc1, ndn1, msg1, idx1, sg1, ss1))

    pltpu.sync_copy(cen_hbm.at[wid], cen_v)
    pltpu.sync_copy(nbr_hbm.at[wid], nbr_v)

    zero16 = jnp.zeros((16,), jnp.float32)

    def _zrow(rr, carry):
        msg0[rr, pl.ds(0, 16)] = zero16
        msg0[rr, pl.ds(TCOLS - 16, 16)] = zero16
        msg1[rr, pl.ds(0, 16)] = zero16
        msg1[rr, pl.ds(TCOLS - 16, 16)] = zero16
        return carry

    lax.fori_loop(0, CHUNK, _zrow, 0)
    for z in range(ROWS_PER_TILE // CHUNK):
        pltpu.sync_copy(
            msg0, acc_s.at[pl.ds(sid * ROWS_PER_TILE + z * CHUNK, CHUNK)])
    plsc.subcore_barrier()

    lane = lax.iota(jnp.int32, 16)

    def _start_gathers(g, p):
        ndc_v, ndn_v, _, _, sg, _ = bufs[p]
        pltpu.async_copy(nt_hbm.at[cen_v.at[g]], ndc_v, sg)
        pltpu.async_copy(nt_hbm.at[nbr_v.at[g]], ndn_v, sg)

    def _wait_gathers(p):
        ndc_v, ndn_v, _, _, sg, _ = bufs[p]
        pltpu.make_async_copy(nt_hbm.at[cen_v.at[0]], ndc_v, sg).wait()
        pltpu.make_async_copy(nt_hbm.at[nbr_v.at[0]], ndn_v, sg).wait()

    def _start_scatter(p):
        _, _, msg_v, idx_v, _, ss = bufs[p]
        pltpu.async_copy(msg_v, acc_s.at[idx_v], ss, add=True)

    def _wait_scatter(p):
        _, _, msg_v, idx_v, _, ss = bufs[p]
        pltpu.make_async_copy(msg_v, acc_s.at[idx_v], ss).wait()

    def _compute_chunk(g, ndc_v, ndn_v, msg_v, idx_v):
        def grp(j, carry):
            row16 = lane + j * 16
            gv = jnp.full((16,), g, jnp.int32)
            c0 = jnp.zeros((16,), jnp.int32)
            cen16 = plsc.load_gather(cen_v, [gv, row16])
            pcx = plsc.load_gather(ndc_v, [row16, c0])
            pcy = plsc.load_gather(ndc_v, [row16, c0 + 1])
            pcz = plsc.load_gather(ndc_v, [row16, c0 + 2])
            pnx = plsc.load_gather(ndn_v, [row16, c0])
            pny = plsc.load_gather(ndn_v, [row16, c0 + 1])
            pnz = plsc.load_gather(ndn_v, [row16, c0 + 2])
            spf = plsc.load_gather(ndn_v, [row16, c0 + 3])

            vx = pnx - pcx
            vy = pny - pcy
            vz = pnz - pcz
            s = vx * vx + vy * vy + vz * vz + 1e-12

            bits = plsc.bitcast(s, jnp.int32)
            y = plsc.bitcast(jnp.int32(0x5F3759DF) - (bits >> 1),
                             jnp.float32)
            hs = 0.5 * s
            for _ in range(3):
                y = y * (1.5 - hs * y * y)
            r = s * y
            ux = vx * y
            uy = vy * y
            uz = vz * y

            xc = jnp.minimum(r * (1.0 / CUTOFF), 1.0)
            u = xc - 0.5
            t = u * u
            sinpu = u * (3.141592653589793 + t * (-5.16771278004997
                    + t * (2.5501640398773455 + t * (-0.5992645293207921
                    + t * (0.08214588661112823
                    + t * -0.007370430945714351)))))
            fc = 0.5 - 0.5 * sinpu

            g0 = jnp.exp(-(r * r)) * fc
            g53 = jnp.exp(-((r - RC_53) * (r - RC_53))) * fc
            g103 = jnp.exp(-((r - RC_103) * (r - RC_103))) * fc
            g25 = jnp.exp(-((r - RC_25) * (r - RC_25))) * fc
            g5 = jnp.exp(-((r - RC_5) * (r - RC_5))) * fc

            sh1 = (uy * SH1, uz * SH1, ux * SH1)
            sh2 = (ux * uy * SH2A, uy * uz * SH2A,
                   (3.0 * uz * uz - 1.0) * SH2C,
                   ux * uz * SH2A, (ux * ux - uy * uy) * SH2E)

            terms = [g0 * SH0, g53 * SH0, g103 * SH0, g5 * SH0]
            rb1 = (g0, g25, g5)
            for m in range(3):
                for n in range(3):
                    terms.append(sh1[m] * rb1[n])
            rb2 = (g0, g5)
            for m in range(5):
                for n in range(2):
                    terms.append(sh2[m] * rb2[n])

            for k, val in enumerate(terms):
                col = jnp.full((16,), k, jnp.int32)
                plsc.store_scatter(msg_v, [row16, col], val)

            idx_v[pl.ds(j * 16, 16)] = cen16 * N_SPEC + spf.astype(jnp.int32)
            return carry

        lax.fori_loop(0, GROUPS, grp, 0)

    _start_gathers(0, 0)

    def _phase(g, p):
        @pl.when(g + 1 < N_CHUNKS)
        def _():
            _start_gathers(g + 1, 1 - p)

        @pl.when(g >= 2)
        def _():
            _wait_scatter(p)

        _wait_gathers(p)
        _compute_chunk(g, bufs[p][0], bufs[p][1], bufs[p][2], bufs[p][3])
        _start_scatter(p)

    def _pipe(g2, carry):
        _phase(g2 * 2, 0)
        _phase(g2 * 2 + 1, 1)
        return carry

    lax.fori_loop(0, N_CHUNKS // 2, _pipe, 0)
    _wait_scatter(0)
    _wait_scatter(1)

    plsc.subcore_barrier()
    rows = pl.ds(sid * ROWS_PER_TILE, ROWS_PER_TILE)
    pltpu.sync_copy(acc_s.at[rows], out_hbm.at[cid, rows, :])


def _sc_call(node_tab, centers, neighbors):
    mesh = plsc.VectorSubcoreMesh(core_axis_name="c", subcore_axis_name="s")
    return pl.kernel(
        _sc_body,
        out_type=jax.ShapeDtypeStruct((2, ACC_ROWS, TCOLS), jnp.float32),
        mesh=mesh,
        compiler_params=pltpu.CompilerParams(
            needs_layout_passes=False, use_tc_tiling_on_sc=False),
        scratch_types=[
            pltpu.VMEM((N_CHUNKS, CHUNK), jnp.int32),
            pltpu.VMEM((N_CHUNKS, CHUNK), jnp.int32),
            pltpu.VMEM((CHUNK, NT_COLS), jnp.float32),
            pltpu.VMEM((CHUNK, NT_COLS), jnp.float32),
            pltpu.VMEM((CHUNK, NT_COLS), jnp.float32),
            pltpu.VMEM((CHUNK, NT_COLS), jnp.float32),
            pltpu.VMEM((CHUNK, TCOLS), jnp.float32),
            pltpu.VMEM((CHUNK, TCOLS), jnp.float32),
            pltpu.VMEM((CHUNK,), jnp.int32),
            pltpu.VMEM((CHUNK,), jnp.int32),
            pltpu.VMEM_SHARED((ACC_ROWS, TCOLS), jnp.float32),
            pltpu.SemaphoreType.DMA,
            pltpu.SemaphoreType.DMA,
            pltpu.SemaphoreType.DMA,
            pltpu.SemaphoreType.DMA,
        ],
    )(node_tab, centers, neighbors)


def _tc_body(p_ref, w_ref, w1_ref, b1_ref, w2_ref, b2_ref, o_ref):
    a = p_ref[0] + p_ref[1]
    feat = jnp.dot(a, w_ref[...], preferred_element_type=jnp.float32,
                   precision=jax.lax.Precision.HIGHEST) * MP
    sq = feat * feat
    inv0 = sq[:, 0:32]
    inv1 = sq[:, 32:56] + sq[:, 56:80] + sq[:, 80:104]
    inv2 = (sq[:, 104:120] + sq[:, 120:136] + sq[:, 136:152]
            + sq[:, 152:168] + sq[:, 168:184])
    inv = jnp.concatenate([inv0, inv1, inv2], axis=1)
    h = jnp.maximum(
        jnp.dot(inv, w1_ref[...], preferred_element_type=jnp.float32)
        + b1_ref[...], 0.0)
    o_ref[...] = (jnp.dot(h, w2_ref[...], preferred_element_type=jnp.float32)
                  + b2_ref[...])


def _tc_head(p, W, W1, b1, W2, b2):
    blk = 1000
    return pl.pallas_call(
        _tc_body,
        grid=(N_NODES // blk,),
        in_specs=[
            pl.BlockSpec((2, blk, N_SPEC * TCOLS), lambda i: (0, i, 0)),
            pl.BlockSpec((N_SPEC * TCOLS, 184), lambda i: (0, 0)),
            pl.BlockSpec((72, 64), lambda i: (0, 0)),
            pl.BlockSpec((1, 64), lambda i: (0, 0)),
            pl.BlockSpec((64, 1), lambda i: (0, 0)),
            pl.BlockSpec((1, 1), lambda i: (0, 0)),
        ],
        out_specs=pl.BlockSpec((blk, 1), lambda i: (i, 0)),
        out_shape=jax.ShapeDtypeStruct((N_NODES, 1), jnp.float32),
    )(p, W, W1, b1, W2, b2)


def _term_onehot():
    import numpy as np
    oh = np.zeros((TCOLS, 8, 184), np.float32)
    k = 0
    for l, (nl, base) in enumerate(((4, 0), (3, 32), (2, 104))):
        for m in range(2 * l + 1):
            for n in range(nl):
                for c in range(8):
                    oh[k, c, base + m * 8 * nl + c * nl + n] = 1.0
                k += 1
    return jnp.asarray(oh)


def kernel(positions, embeddings, W1, b1, W2, b2, species, pairs):
    pp = jnp.pad(pairs.astype(jnp.int32), ((0, E_PAD - N_EDGES), (0, 0)),
                 constant_values=N_NODES)
    centers = pp[:, 0].reshape(N_TILES, N_CHUNKS, CHUNK)
    neighbors = pp[:, 1].reshape(N_TILES, N_CHUNKS, CHUNK)

    node_tab = jnp.pad(
        jnp.concatenate(
            [positions, species.astype(jnp.float32)[:, None]], axis=1),
        ((0, NROWS - N_NODES), (0, NT_COLS - 4)))

    p = _sc_call(node_tab, centers, neighbors)
    p = p.reshape(2, NROWS, N_SPEC * TCOLS)
    W = jnp.einsum('sc,kcj->skj', embeddings, _term_onehot(),
                   precision=jax.lax.Precision.HIGHEST)
    W = W.reshape(N_SPEC * TCOLS, 184)
    return _tc_head(p, W, W1, b1.reshape(1, 64), W2, b2.reshape(1, 1))
